# baseline (speedup 1.0000x reference)
import functools

import jax
import jax.numpy as jnp
from jax import lax
from jax.experimental import pallas as pl
from jax.experimental.pallas import tpu as pltpu
from jax.experimental.pallas import tpu_sc as plsc

EMBED = 64
SEQ = 200
BATCH = 4096
ROWS = BATCH * SEQ
NC, NS = 2, 16
NW = NC * NS
ROWS_PER_W = ROWS // NW
SEQS_PER_W = ROWS_PER_W // SEQ
SCALE = 8.0
GA = 128
GB = SEQ - GA
NBUF = 4


def _sc_embed(idx_flat, table, pos):
  mesh = plsc.VectorSubcoreMesh(core_axis_name="c", subcore_axis_name="s")

  @functools.partial(
      pl.kernel,
      mesh=mesh,
      compiler_params=pltpu.CompilerParams(use_tc_tiling_on_sc=False),
      out_type=jax.ShapeDtypeStruct((BATCH, SEQ, EMBED), jnp.float32),
      scratch_types=[
          pltpu.VMEM((ROWS_PER_W,), jnp.int32),
          pltpu.VMEM((SEQ, EMBED), jnp.float32),
          [pltpu.VMEM((SEQ, EMBED), jnp.float32)] * NBUF,
          [pltpu.SemaphoreType.DMA] * NBUF,
          [pltpu.SemaphoreType.DMA] * NBUF,
      ],
  )
  def k(idx_hbm, table_hbm, pos_hbm, out_hbm, idx_all, pos_v, bufs, gsem, ssem):
    wid = lax.axis_index("s") * NC + lax.axis_index("c")
    base = wid * ROWS_PER_W
    pltpu.sync_copy(idx_hbm.at[pl.ds(base, ROWS_PER_W)], idx_all)
    pltpu.sync_copy(pos_hbm.at[pl.ds(0, SEQ)], pos_v)

    def start_gather(c, b):
      off = c * SEQ
      pltpu.async_copy(table_hbm.at[idx_all.at[pl.ds(off, GA)]],
                       bufs[b].at[pl.ds(0, GA)], gsem[b])
      pltpu.async_copy(table_hbm.at[idx_all.at[pl.ds(off + GA, GB)]],
                       bufs[b].at[pl.ds(GA, GB)], gsem[b])

    def wait_gather(b):
      pltpu.make_async_copy(table_hbm.at[idx_all.at[pl.ds(0, GA)]],
                            bufs[b].at[pl.ds(0, GA)], gsem[b]).wait()
      pltpu.make_async_copy(table_hbm.at[idx_all.at[pl.ds(0, GB)]],
                            bufs[b].at[pl.ds(GA, GB)], gsem[b]).wait()

    def start_scatter(c, b):
      pltpu.async_copy(bufs[b], out_hbm.at[wid * SEQS_PER_W + c], ssem[b])

    def wait_scatter(b):
      pltpu.make_async_copy(bufs[b], out_hbm.at[0], ssem[b]).wait()

    def compute(b):
      buf = bufs[b]

      def row_body(r, rc):
        for q in range(EMBED // 16):
          sl = pl.ds(q * 16, 16)
          buf[r, sl] = buf[r, sl] * SCALE + pos_v[r, sl]
        return rc

      lax.fori_loop(0, SEQ, row_body, 0, unroll=4)

    start_gather(0, 0)
    start_gather(1, 1)

    def step(i, carry):
      for b in range(NBUF):
        c = i * NBUF + b
        wait_gather(b)
        compute(b)
        nb = (b + 2) % NBUF

        @pl.when(c >= 2)
        def _():
          wait_scatter(nb)

        @pl.when(c + 2 < SEQS_PER_W)
        def _():
          start_gather(c + 2, nb)

        start_scatter(c, b)
      return carry

    lax.fori_loop(0, SEQS_PER_W // NBUF, step, 0)
    wait_scatter((SEQS_PER_W - 2) % NBUF)
    wait_scatter((SEQS_PER_W - 1) % NBUF)

  return k(idx_flat, table, pos)


def kernel(input_tensor, src_table, pos_table):
  idx_flat = input_tensor.reshape(ROWS).astype(jnp.int32)
  return _sc_embed(idx_flat, src_table, pos_table)


# speedup vs baseline: 2.0341x; 2.0341x over previous
import functools

import jax
import jax.numpy as jnp
from jax import lax
from jax.experimental import pallas as pl
from jax.experimental.pallas import tpu as pltpu
from jax.experimental.pallas import tpu_sc as plsc

EMBED = 64
SEQ = 200
BATCH = 4096
ROWS = BATCH * SEQ
MID_W = 128
NC, NS = 2, 16
NW = NC * NS
BPW = BATCH // NW
SCALE = 8.0
NBUF = 4
BBLK = 4096


def _sc_gather(idx_t, table):
  mesh = plsc.VectorSubcoreMesh(core_axis_name="c", subcore_axis_name="s")

  @functools.partial(
      pl.kernel,
      mesh=mesh,
      compiler_params=pltpu.CompilerParams(use_tc_tiling_on_sc=False),
      out_type=jax.ShapeDtypeStruct((ROWS, MID_W), jnp.float32),
      scratch_types=[
          pltpu.VMEM((SEQ, BPW), jnp.int32),
          [pltpu.VMEM((BPW, EMBED), jnp.float32)] * NBUF,
          [pltpu.SemaphoreType.DMA] * NBUF,
          [pltpu.SemaphoreType.DMA] * NBUF,
      ],
  )
  def k(idx_hbm, table_hbm, mid_hbm, idx_v, bufs, gsem, ssem):
    wid = lax.axis_index("s") * NC + lax.axis_index("c")
    b0 = wid * BPW
    pltpu.sync_copy(idx_hbm.at[:, pl.ds(b0, BPW)], idx_v)

    def start_gather(s, b):
      pltpu.async_copy(table_hbm.at[idx_v.at[s]], bufs[b], gsem[b])

    def wait_gather(b):
      pltpu.make_async_copy(table_hbm.at[idx_v.at[0]], bufs[b], gsem[b]).wait()

    def start_scatter(s, b):
      pltpu.async_copy(
          bufs[b],
          mid_hbm.at[pl.ds(s * BATCH + b0, BPW), pl.ds(0, EMBED)], ssem[b])

    def wait_scatter(b):
      pltpu.make_async_copy(
          bufs[b], mid_hbm.at[pl.ds(0, BPW), pl.ds(0, EMBED)], ssem[b]).wait()

    start_gather(0, 0)
    start_gather(1, 1)

    def step(i, carry):
      for b in range(NBUF):
        s = i * NBUF + b
        wait_gather(b)
        nb = (b + 2) % NBUF

        @pl.when(s >= 2)
        def _():
          wait_scatter(nb)

        @pl.when(s + 2 < SEQ)
        def _():
          start_gather(s + 2, nb)

        start_scatter(s, b)
      return carry

    lax.fori_loop(0, SEQ // NBUF, step, 0)
    wait_scatter((SEQ - 2) % NBUF)
    wait_scatter((SEQ - 1) % NBUF)

  return k(idx_t, table)


def _tc_finish(mid3, pos):
  def body(in_ref, pos_ref, out_ref):
    x = in_ref[0]
    v = x[:, :EMBED]
    p = pos_ref[pl.ds(pl.program_id(0), 1), :]
    out_ref[0] = v.T * SCALE + p.T

  return pl.pallas_call(
      body,
      grid=(SEQ, BATCH // BBLK),
      in_specs=[
          pl.BlockSpec((1, BBLK, MID_W), lambda s, j: (s, j, 0)),
          pl.BlockSpec((512, EMBED), lambda s, j: (0, 0)),
      ],
      out_specs=pl.BlockSpec((1, EMBED, BBLK), lambda s, j: (s, 0, j)),
      out_shape=jax.ShapeDtypeStruct((SEQ, EMBED, BATCH), jnp.float32),
  )(mid3, pos)


def kernel(input_tensor, src_table, pos_table):
  idx_t = input_tensor.T.astype(jnp.int32)
  mid = _sc_gather(idx_t, src_table)
  mid3 = mid.reshape(SEQ, BATCH, MID_W)
  out_t = _tc_finish(mid3, pos_table)
  return jnp.transpose(out_t, (2, 0, 1))


# speedup vs baseline: 2.1138x; 1.0392x over previous
import functools

import jax
import jax.numpy as jnp
from jax import lax
from jax.experimental import pallas as pl
from jax.experimental.pallas import tpu as pltpu
from jax.experimental.pallas import tpu_sc as plsc

EMBED = 64
SEQ = 200
BATCH = 4096
ROWS = BATCH * SEQ
MID_W = 128
NC, NS = 2, 16
NW = NC * NS
BPW = BATCH // NW
SCALE = 8.0
NBUF = 4
BBLK = 4096


def _sc_gather(idx_t, table):
  mesh = plsc.VectorSubcoreMesh(core_axis_name="c", subcore_axis_name="s")

  @functools.partial(
      pl.kernel,
      mesh=mesh,
      compiler_params=pltpu.CompilerParams(use_tc_tiling_on_sc=False),
      out_type=jax.ShapeDtypeStruct((ROWS, MID_W), jnp.float32),
      scratch_types=[
          pltpu.VMEM((SEQ, BPW), jnp.int32),
          [pltpu.VMEM((BPW, EMBED), jnp.float32)] * NBUF,
          [pltpu.SemaphoreType.DMA] * NBUF,
          [pltpu.SemaphoreType.DMA] * NBUF,
      ],
  )
  def k(idx_hbm, table_hbm, mid_hbm, idx_v, bufs, gsem, ssem):
    wid = lax.axis_index("s") * NC + lax.axis_index("c")
    b0 = wid * BPW
    pltpu.sync_copy(idx_hbm.at[:, pl.ds(b0, BPW)], idx_v)

    def start_gather(s, b):
      pltpu.async_copy(table_hbm.at[idx_v.at[s]], bufs[b], gsem[b])

    def wait_gather(b):
      pltpu.make_async_copy(table_hbm.at[idx_v.at[0]], bufs[b], gsem[b]).wait()

    def start_scatter(s, b):
      pltpu.async_copy(
          bufs[b],
          mid_hbm.at[pl.ds(s * BATCH + b0, BPW), pl.ds(0, EMBED)], ssem[b])

    def wait_scatter(b):
      pltpu.make_async_copy(
          bufs[b], mid_hbm.at[pl.ds(0, BPW), pl.ds(0, EMBED)], ssem[b]).wait()

    start_gather(0, 0)
    start_gather(1, 1)

    def step(i, carry):
      for b in range(NBUF):
        s = i * NBUF + b
        wait_gather(b)
        nb = (b + 2) % NBUF

        @pl.when(s >= 2)
        def _():
          wait_scatter(nb)

        @pl.when(s + 2 < SEQ)
        def _():
          start_gather(s + 2, nb)

        start_scatter(s, b)
      return carry

    lax.fori_loop(0, SEQ // NBUF, step, 0)
    wait_scatter((SEQ - 2) % NBUF)
    wait_scatter((SEQ - 1) % NBUF)

  return k(idx_t, table)


def _tc_finish(mid3, pos):
  def body(in_ref, pos_ref, out_ref):
    x = in_ref[0]
    v = x[:, :EMBED]
    r = lax.broadcasted_iota(jnp.int32, (EMBED, EMBED), 0)
    c = lax.broadcasted_iota(jnp.int32, (EMBED, EMBED), 1)
    eye = jnp.where(r == c, SCALE, 0.0).astype(jnp.float32)
    y = lax.dot_general(eye, v, (((1,), (1,)), ((), ())),
                        preferred_element_type=jnp.float32)
    p = pos_ref[pl.ds(pl.program_id(0), 1), :]
    out_ref[0] = y + p.T

  return pl.pallas_call(
      body,
      grid=(SEQ, BATCH // BBLK),
      in_specs=[
          pl.BlockSpec((1, BBLK, MID_W), lambda s, j: (s, j, 0)),
          pl.BlockSpec((512, EMBED), lambda s, j: (0, 0)),
      ],
      out_specs=pl.BlockSpec((1, EMBED, BBLK), lambda s, j: (s, 0, j)),
      out_shape=jax.ShapeDtypeStruct((SEQ, EMBED, BATCH), jnp.float32),
  )(mid3, pos)


def kernel(input_tensor, src_table, pos_table):
  idx_t = input_tensor.T.astype(jnp.int32)
  mid = _sc_gather(idx_t, src_table)
  mid3 = mid.reshape(SEQ, BATCH, MID_W)
  out_t = _tc_finish(mid3, pos_table)
  return jnp.transpose(out_t, (2, 0, 1))
